# baseline (speedup 1.0000x reference)
import numpy as np
import jax
import jax.numpy as jnp
from jax.experimental import pallas as pl
from jax.experimental.pallas import tpu as pltpu

_C = 96
_HW = 28
_NPOS = _HW * _HW
_KA = 7
_NS = 500
_NW = 9
_NLOC = _C // 2
_G = 16


def _threefry_bits(seed, total):
    rot = [[13, 15, 26, 6], [17, 29, 16, 24]]
    ks = [np.uint32(0), np.uint32(seed),
          np.uint32(np.uint32(0) ^ np.uint32(seed) ^ np.uint32(0x1BD11BDA))]
    x0 = np.full(total, ks[0], np.uint32)
    x1 = (np.arange(total, dtype=np.uint32) + ks[1]).astype(np.uint32)

    def rotl(x, d):
        return ((x << np.uint32(d)) | (x >> np.uint32(32 - d))).astype(np.uint32)

    for i in range(5):
        for r in rot[i % 2]:
            x0 = (x0 + x1).astype(np.uint32)
            x1 = rotl(x1, r)
            x1 = x1 ^ x0
        x0 = (x0 + ks[(i + 1) % 3]).astype(np.uint32)
        x1 = (x1 + ks[(i + 2) % 3] + np.uint32(i + 1)).astype(np.uint32)
    return x0 ^ x1


def _erfinv_f32(x):
    x = x.astype(np.float32)
    w = (-np.log(((np.float32(1.0) - x) * (np.float32(1.0) + x))
                 .astype(np.float32)).astype(np.float32))
    small = w < np.float32(5.0)
    ws = (w - np.float32(2.5)).astype(np.float32)
    wb = (np.sqrt(np.maximum(w, np.float32(5.0))).astype(np.float32)
          - np.float32(3.0)).astype(np.float32)
    ca = [2.81022636e-08, 3.43273939e-07, -3.5233877e-06, -4.39150654e-06,
          0.00021858087, -0.00125372503, -0.00417768164, 0.246640727,
          1.50140941]
    cb = [-0.000200214257, 0.000100950558, 0.00134934322, -0.00367342844,
          0.00573950773, -0.0076224613, 0.00943887047, 1.00167406,
          2.83297682]
    pa = np.full_like(ws, np.float32(ca[0]))
    for c in ca[1:]:
        pa = (pa * ws + np.float32(c)).astype(np.float32)
    pb = np.full_like(wb, np.float32(cb[0]))
    for c in cb[1:]:
        pb = (pb * wb + np.float32(c)).astype(np.float32)
    return (np.where(small, pa, pb) * x).astype(np.float32)


def _noise_t(n):
    total = n * _NS * _NW
    bits = _threefry_bits(1, total)
    fb = ((bits >> np.uint32(9)) | np.uint32(0x3F800000)).view(np.float32)
    u01 = fb - np.float32(1.0)
    lo = np.float32(np.nextafter(np.float32(-1.0), np.float32(0.0)))
    u = np.maximum(lo, (u01 * (np.float32(1.0) - lo) + lo).astype(np.float32))
    z = (np.float32(np.sqrt(2.0)) * _erfinv_f32(u)).astype(np.float32)
    return z.reshape(n, _NS, _NW).transpose(0, 2, 1).copy()


def _gelu(x):
    return x * 0.5 * (1.0 + jax.lax.erf(x * np.float32(1.0 / np.sqrt(2.0))))


def _masks():
    wm = np.zeros((_NPOS, _NW), np.float32)
    km = np.zeros((_NPOS, _KA * _KA), np.float32)
    for r in range(3):
        for c in range(3):
            j = r * 3 + c
            for dr in range(_KA):
                for dc in range(_KA):
                    p = (10 * r + dr) * _HW + (10 * c + dc)
                    wm[p, j] = 1.0 / (_KA * _KA)
                    km[p, dr * _KA + dc] = 1.0
    return wm, km


def _body(x_ref, noise_ref, sig_ref, g_ref, lnb_ref, w1_ref, b1_ref,
          w2_ref, b2_ref, wm_ref, km_ref, out_ref):
    f32 = jnp.float32
    o96 = jnp.ones((1, _C), f32)
    o784 = jnp.ones((_NPOS, 1), f32)
    c_isq2 = np.float32(1.0 / np.sqrt(2.0))
    ct = (((0,), (0,)), ((), ()))
    w1g = w1_ref[...] * g_ref[...]
    w1gs = jnp.dot(o96, w1g, preferred_element_type=f32)
    hb = (jax.lax.dot_general(w1_ref[...], lnb_ref[...], ct,
                              preferred_element_type=f32)
          + b1_ref[...])
    w2l = w2_ref[:_NLOC] * 0.5
    w2g = w2_ref[_NLOC:] * 0.5
    patches = []
    for gidx in range(_G):
        xc = x_ref[0, gidx]
        m = jnp.dot(o96, xc, preferred_element_type=f32) * (1.0 / _C)
        q = jnp.dot(o96, xc * xc, preferred_element_type=f32) * (1.0 / _C)
        inv = jax.lax.rsqrt(q - m * m + 1e-5)
        he = jax.lax.dot_general(w1g, xc, ct,
                                 preferred_element_type=f32)
        y = (he - jax.lax.dot_general(w1gs, m, ct,
                                      preferred_element_type=f32)) * inv + hb
        h = y * (1.0 + jax.lax.erf(y * c_isq2))
        glb = jnp.dot(h[_NLOC:, :], o784,
                      preferred_element_type=f32) * (1.0 / _NPOS)
        s = jax.lax.dot_general(w2l, h[:_NLOC, :], ct,
                                preferred_element_type=f32)
        gg = jax.lax.dot_general(w2g, glb, ct,
                                 preferred_element_type=f32)
        s = _gelu(s + gg + b2_ref[...])
        ws = jax.lax.dot_general(wm_ref[...], s, (((0,), (1,)), ((), ())),
                                 preferred_element_type=f32)
        mn = jnp.min(ws, axis=0, keepdims=True)
        mx = jnp.max(ws, axis=0, keepdims=True)
        wsn = (ws - mn) / (mx - mn + 1e-5)
        pert = wsn + noise_ref[0, gidx] * sig_ref[...]
        iota = jax.lax.broadcasted_iota(jnp.int32, (_NW, _NS), 0)
        cmax = jnp.max(pert, axis=0, keepdims=True)
        idx = jnp.min(jnp.where(pert == cmax, iota, _NW), axis=0,
                      keepdims=True)
        ind = jnp.sum((iota == idx).astype(f32), axis=1,
                      keepdims=True) * (1.0 / _NS)
        indrow = jax.lax.dot_general(ind * f32(_KA * _KA), wm_ref[...],
                                     (((0,), (1,)), ((), ())),
                                     preferred_element_type=f32)
        patches.append(jnp.dot(xc * indrow, km_ref[...],
                               preferred_element_type=f32))
    out_ref[0] = jnp.concatenate(patches, axis=1)


def kernel(x, type, H, W, T, sigma, ln_g, ln_b, w1, b1, w2, b2):
    B, Ts = x.shape[0], x.shape[1]
    n = B * Ts
    x4 = x.reshape(B, Ts, _C, _NPOS)
    noise_t = jnp.asarray(_noise_t(n).reshape(B, Ts, _NW, _NS))
    sig = jnp.asarray(sigma, jnp.float32).reshape(1, 1)
    wm, km = _masks()
    out = pl.pallas_call(
        _body,
        grid=(B, Ts // _G),
        in_specs=[
            pl.BlockSpec((1, _G, _C, _NPOS), lambda b, t: (b, t, 0, 0)),
            pl.BlockSpec((1, _G, _NW, _NS), lambda b, t: (b, t, 0, 0)),
            pl.BlockSpec((1, 1), lambda b, t: (0, 0)),
            pl.BlockSpec((_C, 1), lambda b, t: (0, 0)),
            pl.BlockSpec((_C, 1), lambda b, t: (0, 0)),
            pl.BlockSpec((_C, _C), lambda b, t: (0, 0)),
            pl.BlockSpec((_C, 1), lambda b, t: (0, 0)),
            pl.BlockSpec((_C, 1), lambda b, t: (0, 0)),
            pl.BlockSpec((1, 1), lambda b, t: (0, 0)),
            pl.BlockSpec((_NPOS, _NW), lambda b, t: (0, 0)),
            pl.BlockSpec((_NPOS, _KA * _KA), lambda b, t: (0, 0)),
        ],
        out_specs=pl.BlockSpec((1, _C, _G * _KA * _KA),
                               lambda b, t: (b, 0, t)),
        out_shape=jax.ShapeDtypeStruct((B, _C, Ts * _KA * _KA), jnp.float32),
        compiler_params=pltpu.CompilerParams(
            dimension_semantics=("parallel", "parallel")),
    )(x4, noise_t, sig, ln_g.reshape(_C, 1), ln_b.reshape(_C, 1), w1,
      b1.reshape(_C, 1), w2.reshape(_C, 1), b2.reshape(1, 1),
      jnp.asarray(wm), jnp.asarray(km))
    return out.reshape(B, _C, Ts, _KA, _KA)


# speedup vs baseline: 1.0379x; 1.0379x over previous
import numpy as np
import jax
import jax.numpy as jnp
from jax.experimental import pallas as pl
from jax.experimental.pallas import tpu as pltpu

_C = 96
_HW = 28
_NPOS = _HW * _HW
_KA = 7
_NS = 500
_NW = 9
_NLOC = _C // 2
_G = 16


def _threefry_bits(seed, total):
    rot = [[13, 15, 26, 6], [17, 29, 16, 24]]
    ks = [np.uint32(0), np.uint32(seed),
          np.uint32(np.uint32(0) ^ np.uint32(seed) ^ np.uint32(0x1BD11BDA))]
    x0 = np.full(total, ks[0], np.uint32)
    x1 = (np.arange(total, dtype=np.uint32) + ks[1]).astype(np.uint32)

    def rotl(x, d):
        return ((x << np.uint32(d)) | (x >> np.uint32(32 - d))).astype(np.uint32)

    for i in range(5):
        for r in rot[i % 2]:
            x0 = (x0 + x1).astype(np.uint32)
            x1 = rotl(x1, r)
            x1 = x1 ^ x0
        x0 = (x0 + ks[(i + 1) % 3]).astype(np.uint32)
        x1 = (x1 + ks[(i + 2) % 3] + np.uint32(i + 1)).astype(np.uint32)
    return x0 ^ x1


def _erfinv_f32(x):
    x = x.astype(np.float32)
    w = (-np.log(((np.float32(1.0) - x) * (np.float32(1.0) + x))
                 .astype(np.float32)).astype(np.float32))
    small = w < np.float32(5.0)
    ws = (w - np.float32(2.5)).astype(np.float32)
    wb = (np.sqrt(np.maximum(w, np.float32(5.0))).astype(np.float32)
          - np.float32(3.0)).astype(np.float32)
    ca = [2.81022636e-08, 3.43273939e-07, -3.5233877e-06, -4.39150654e-06,
          0.00021858087, -0.00125372503, -0.00417768164, 0.246640727,
          1.50140941]
    cb = [-0.000200214257, 0.000100950558, 0.00134934322, -0.00367342844,
          0.00573950773, -0.0076224613, 0.00943887047, 1.00167406,
          2.83297682]
    pa = np.full_like(ws, np.float32(ca[0]))
    for c in ca[1:]:
        pa = (pa * ws + np.float32(c)).astype(np.float32)
    pb = np.full_like(wb, np.float32(cb[0]))
    for c in cb[1:]:
        pb = (pb * wb + np.float32(c)).astype(np.float32)
    return (np.where(small, pa, pb) * x).astype(np.float32)


def _noise_t(n):
    total = n * _NS * _NW
    bits = _threefry_bits(1, total)
    fb = ((bits >> np.uint32(9)) | np.uint32(0x3F800000)).view(np.float32)
    u01 = fb - np.float32(1.0)
    lo = np.float32(np.nextafter(np.float32(-1.0), np.float32(0.0)))
    u = np.maximum(lo, (u01 * (np.float32(1.0) - lo) + lo).astype(np.float32))
    z = (np.float32(np.sqrt(2.0)) * _erfinv_f32(u)).astype(np.float32)
    return z.reshape(n, _NS, _NW).transpose(0, 2, 1).copy()


def _gelu(x):
    return x * 0.5 * (1.0 + jax.lax.erf(x * np.float32(1.0 / np.sqrt(2.0))))


def _masks():
    wm = np.zeros((_NPOS, _NW), np.float32)
    km = np.zeros((_NPOS, _KA * _KA), np.float32)
    for r in range(3):
        for c in range(3):
            j = r * 3 + c
            for dr in range(_KA):
                for dc in range(_KA):
                    p = (10 * r + dr) * _HW + (10 * c + dc)
                    wm[p, j] = 1.0 / (_KA * _KA)
                    km[p, dr * _KA + dc] = 1.0
    return wm, km


def _body(x_ref, noise_ref, sig_ref, w1e_ref, w1gs_ref, hb_ref,
          w2l_ref, w2g_ref, b2_ref, wm_ref, km_ref, out_ref):
    f32 = jnp.float32
    o96 = jnp.ones((1, _C), f32)
    o784 = jnp.ones((_NPOS, 1), f32)
    c_isq2 = np.float32(1.0 / np.sqrt(2.0))
    patches = []
    for gidx in range(_G):
        xc = x_ref[0, gidx]
        he = jnp.dot(w1e_ref[...], xc, preferred_element_type=f32)
        m = he[_C:_C + 1, :] * (1.0 / _C)
        q = jnp.dot(o96, xc * xc, preferred_element_type=f32) * (1.0 / _C)
        inv = jax.lax.rsqrt(q - m * m + 1e-5)
        y = (he[:_C, :] - w1gs_ref[...] * m) * inv + hb_ref[...]
        h = y * (1.0 + jax.lax.erf(y * c_isq2))
        glb = jnp.dot(h[_NLOC:, :], o784,
                      preferred_element_type=f32) * (1.0 / _NPOS)
        s = jax.lax.dot_general(w2l_ref[...], h[:_NLOC, :],
                                (((0,), (0,)), ((), ())),
                                preferred_element_type=f32)
        gg = jax.lax.dot_general(w2g_ref[...], glb, (((0,), (0,)), ((), ())),
                                 preferred_element_type=f32)
        s = _gelu(s + gg + b2_ref[...])
        ws = jax.lax.dot_general(wm_ref[...], s, (((0,), (1,)), ((), ())),
                                 preferred_element_type=f32)
        mn = jnp.min(ws, axis=0, keepdims=True)
        mx = jnp.max(ws, axis=0, keepdims=True)
        wsn = (ws - mn) / (mx - mn + 1e-5)
        pert = wsn + noise_ref[0, gidx] * sig_ref[...]
        iota = jax.lax.broadcasted_iota(jnp.int32, (_NW, _NS), 0)
        cmax = jnp.max(pert, axis=0, keepdims=True)
        idx = jnp.min(jnp.where(pert == cmax, iota, _NW), axis=0,
                      keepdims=True)
        ind = jnp.sum((iota == idx).astype(f32), axis=1,
                      keepdims=True) * (1.0 / _NS)
        indrow = jax.lax.dot_general(ind * f32(_KA * _KA), wm_ref[...],
                                     (((0,), (1,)), ((), ())),
                                     preferred_element_type=f32)
        patches.append(jnp.dot(xc * indrow, km_ref[...],
                               preferred_element_type=f32))
    out_ref[0] = jnp.concatenate(patches, axis=1)


def kernel(x, type, H, W, T, sigma, ln_g, ln_b, w1, b1, w2, b2):
    B, Ts = x.shape[0], x.shape[1]
    n = B * Ts
    x4 = x.reshape(B, Ts, _C, _NPOS)
    noise_t = jnp.asarray(_noise_t(n).reshape(B, Ts, _NW, _NS))
    sig = jnp.asarray(sigma, jnp.float32).reshape(1, 1)
    wm, km = _masks()
    w1g = w1.T * ln_g[None, :]
    w1gs = jnp.sum(w1g, axis=1, keepdims=True)
    w1e = jnp.zeros((_C + 8, _C), jnp.float32)
    w1e = w1e.at[:_C].set(w1g).at[_C].set(1.0)
    hb = (w1.T @ ln_b + b1).reshape(_C, 1)
    out = pl.pallas_call(
        _body,
        grid=(B, Ts // _G),
        in_specs=[
            pl.BlockSpec((1, _G, _C, _NPOS), lambda b, t: (b, t, 0, 0)),
            pl.BlockSpec((1, _G, _NW, _NS), lambda b, t: (b, t, 0, 0)),
            pl.BlockSpec((1, 1), lambda b, t: (0, 0)),
            pl.BlockSpec((_C + 8, _C), lambda b, t: (0, 0)),
            pl.BlockSpec((_C, 1), lambda b, t: (0, 0)),
            pl.BlockSpec((_C, 1), lambda b, t: (0, 0)),
            pl.BlockSpec((_NLOC, 1), lambda b, t: (0, 0)),
            pl.BlockSpec((_NLOC, 1), lambda b, t: (0, 0)),
            pl.BlockSpec((1, 1), lambda b, t: (0, 0)),
            pl.BlockSpec((_NPOS, _NW), lambda b, t: (0, 0)),
            pl.BlockSpec((_NPOS, _KA * _KA), lambda b, t: (0, 0)),
        ],
        out_specs=pl.BlockSpec((1, _C, _G * _KA * _KA),
                               lambda b, t: (b, 0, t)),
        out_shape=jax.ShapeDtypeStruct((B, _C, Ts * _KA * _KA), jnp.float32),
        compiler_params=pltpu.CompilerParams(
            dimension_semantics=("parallel", "parallel")),
    )(x4, noise_t, sig, w1e, w1gs, hb,
      0.5 * w2[:_NLOC].reshape(_NLOC, 1), 0.5 * w2[_NLOC:].reshape(_NLOC, 1),
      b2.reshape(1, 1), jnp.asarray(wm), jnp.asarray(km))
    return out.reshape(B, _C, Ts, _KA, _KA)


# speedup vs baseline: 1.0439x; 1.0058x over previous
import numpy as np
import jax
import jax.numpy as jnp
from jax.experimental import pallas as pl
from jax.experimental.pallas import tpu as pltpu

_C = 96
_HW = 28
_NPOS = _HW * _HW
_KA = 7
_NS = 500
_NW = 9
_NLOC = _C // 2
_G = 16


def _threefry_bits(seed, total):
    rot = [[13, 15, 26, 6], [17, 29, 16, 24]]
    ks = [np.uint32(0), np.uint32(seed),
          np.uint32(np.uint32(0) ^ np.uint32(seed) ^ np.uint32(0x1BD11BDA))]
    x0 = np.full(total, ks[0], np.uint32)
    x1 = (np.arange(total, dtype=np.uint32) + ks[1]).astype(np.uint32)

    def rotl(x, d):
        return ((x << np.uint32(d)) | (x >> np.uint32(32 - d))).astype(np.uint32)

    for i in range(5):
        for r in rot[i % 2]:
            x0 = (x0 + x1).astype(np.uint32)
            x1 = rotl(x1, r)
            x1 = x1 ^ x0
        x0 = (x0 + ks[(i + 1) % 3]).astype(np.uint32)
        x1 = (x1 + ks[(i + 2) % 3] + np.uint32(i + 1)).astype(np.uint32)
    return x0 ^ x1


def _erfinv_f32(x):
    x = x.astype(np.float32)
    w = (-np.log(((np.float32(1.0) - x) * (np.float32(1.0) + x))
                 .astype(np.float32)).astype(np.float32))
    small = w < np.float32(5.0)
    ws = (w - np.float32(2.5)).astype(np.float32)
    wb = (np.sqrt(np.maximum(w, np.float32(5.0))).astype(np.float32)
          - np.float32(3.0)).astype(np.float32)
    ca = [2.81022636e-08, 3.43273939e-07, -3.5233877e-06, -4.39150654e-06,
          0.00021858087, -0.00125372503, -0.00417768164, 0.246640727,
          1.50140941]
    cb = [-0.000200214257, 0.000100950558, 0.00134934322, -0.00367342844,
          0.00573950773, -0.0076224613, 0.00943887047, 1.00167406,
          2.83297682]
    pa = np.full_like(ws, np.float32(ca[0]))
    for c in ca[1:]:
        pa = (pa * ws + np.float32(c)).astype(np.float32)
    pb = np.full_like(wb, np.float32(cb[0]))
    for c in cb[1:]:
        pb = (pb * wb + np.float32(c)).astype(np.float32)
    return (np.where(small, pa, pb) * x).astype(np.float32)


def _noise_t(n):
    total = n * _NS * _NW
    bits = _threefry_bits(1, total)
    fb = ((bits >> np.uint32(9)) | np.uint32(0x3F800000)).view(np.float32)
    u01 = fb - np.float32(1.0)
    lo = np.float32(np.nextafter(np.float32(-1.0), np.float32(0.0)))
    u = np.maximum(lo, (u01 * (np.float32(1.0) - lo) + lo).astype(np.float32))
    z = (np.float32(np.sqrt(2.0)) * _erfinv_f32(u)).astype(np.float32)
    return z.reshape(n, _NS, _NW).transpose(0, 2, 1).copy()


def _gelu(x):
    return x * 0.5 * (1.0 + jax.lax.erf(x * np.float32(1.0 / np.sqrt(2.0))))


def _masks():
    wm = np.zeros((_NPOS, _NW), np.float32)
    km = np.zeros((_NPOS, _KA * _KA), np.float32)
    for r in range(3):
        for c in range(3):
            j = r * 3 + c
            for dr in range(_KA):
                for dc in range(_KA):
                    p = (10 * r + dr) * _HW + (10 * c + dc)
                    wm[p, j] = 1.0 / (_KA * _KA)
                    km[p, dr * _KA + dc] = 1.0
    return wm, km


def _body(x_ref, noise_ref, sig_ref, w1e_ref, hb_ref,
          w2l_ref, w2g_ref, b2_ref, wm_ref, km_ref, out_ref):
    f32 = jnp.float32
    o96 = jnp.ones((1, _C), f32)
    o784 = jnp.ones((_NPOS, 1), f32)
    c_isq2 = np.float32(1.0 / np.sqrt(2.0))
    patches = []
    for gidx in range(_G):
        xc = x_ref[0, gidx]
        he = jnp.dot(w1e_ref[...], xc, preferred_element_type=f32)
        m = he[_C:_C + 1, :] * (1.0 / _C)
        q = jnp.dot(o96, xc * xc, preferred_element_type=f32) * (1.0 / _C)
        inv = jax.lax.rsqrt(q - m * m + 1e-5)
        y = he[:_C, :] * inv + hb_ref[...]
        h = y * (1.0 + jax.lax.erf(y * c_isq2))
        glb = jnp.dot(h[_NLOC:, :], o784,
                      preferred_element_type=f32) * (1.0 / _NPOS)
        s = jax.lax.dot_general(w2l_ref[...], h[:_NLOC, :],
                                (((0,), (0,)), ((), ())),
                                preferred_element_type=f32)
        gg = jax.lax.dot_general(w2g_ref[...], glb, (((0,), (0,)), ((), ())),
                                 preferred_element_type=f32)
        s = _gelu(s + gg + b2_ref[...])
        ws = jax.lax.dot_general(wm_ref[...], s, (((0,), (1,)), ((), ())),
                                 preferred_element_type=f32)
        mn = jnp.min(ws, axis=0, keepdims=True)
        mx = jnp.max(ws, axis=0, keepdims=True)
        wsn = (ws - mn) / (mx - mn + 1e-5)
        pert = wsn + noise_ref[0, gidx] * sig_ref[...]
        iota = jax.lax.broadcasted_iota(jnp.int32, (_NW, _NS), 0)
        cmax = jnp.max(pert, axis=0, keepdims=True)
        idx = jnp.min(jnp.where(pert == cmax, iota, _NW), axis=0,
                      keepdims=True)
        ind = jnp.sum((iota == idx).astype(f32), axis=1,
                      keepdims=True) * (1.0 / _NS)
        indrow = jax.lax.dot_general(ind * f32(_KA * _KA), wm_ref[...],
                                     (((0,), (1,)), ((), ())),
                                     preferred_element_type=f32)
        patches.append(jnp.dot(xc * indrow, km_ref[...],
                               preferred_element_type=f32))
    out_ref[0] = jnp.concatenate(patches, axis=1)


def kernel(x, type, H, W, T, sigma, ln_g, ln_b, w1, b1, w2, b2):
    B, Ts = x.shape[0], x.shape[1]
    n = B * Ts
    x4 = x.reshape(B, Ts, _C, _NPOS)
    noise_t = jnp.asarray(_noise_t(n).reshape(B, Ts, _NW, _NS))
    sig = jnp.asarray(sigma, jnp.float32).reshape(1, 1)
    wm, km = _masks()
    w1g = w1.T * ln_g[None, :]
    w1gc = w1g - jnp.sum(w1g, axis=1, keepdims=True) * (1.0 / _C)
    w1e = jnp.zeros((_C + 8, _C), jnp.float32)
    w1e = w1e.at[:_C].set(w1gc).at[_C].set(1.0)
    hb = (w1.T @ ln_b + b1).reshape(_C, 1)
    out = pl.pallas_call(
        _body,
        grid=(B, Ts // _G),
        in_specs=[
            pl.BlockSpec((1, _G, _C, _NPOS), lambda b, t: (b, t, 0, 0)),
            pl.BlockSpec((1, _G, _NW, _NS), lambda b, t: (b, t, 0, 0)),
            pl.BlockSpec((1, 1), lambda b, t: (0, 0)),
            pl.BlockSpec((_C + 8, _C), lambda b, t: (0, 0)),
            pl.BlockSpec((_C, 1), lambda b, t: (0, 0)),
            pl.BlockSpec((_NLOC, 1), lambda b, t: (0, 0)),
            pl.BlockSpec((_NLOC, 1), lambda b, t: (0, 0)),
            pl.BlockSpec((1, 1), lambda b, t: (0, 0)),
            pl.BlockSpec((_NPOS, _NW), lambda b, t: (0, 0)),
            pl.BlockSpec((_NPOS, _KA * _KA), lambda b, t: (0, 0)),
        ],
        out_specs=pl.BlockSpec((1, _C, _G * _KA * _KA),
                               lambda b, t: (b, 0, t)),
        out_shape=jax.ShapeDtypeStruct((B, _C, Ts * _KA * _KA), jnp.float32),
        compiler_params=pltpu.CompilerParams(
            dimension_semantics=("parallel", "parallel")),
    )(x4, noise_t, sig, w1e, hb,
      0.5 * w2[:_NLOC].reshape(_NLOC, 1), 0.5 * w2[_NLOC:].reshape(_NLOC, 1),
      b2.reshape(1, 1), jnp.asarray(wm), jnp.asarray(km))
    return out.reshape(B, _C, Ts, _KA, _KA)


# speedup vs baseline: 1.4122x; 1.3528x over previous
import numpy as np
import jax
import jax.numpy as jnp
from jax.experimental import pallas as pl
from jax.experimental.pallas import tpu as pltpu

_C = 96
_HW = 28
_NPOS = _HW * _HW
_KA = 7
_NS = 500
_NW = 9
_NLOC = _C // 2
_G = 16


def _threefry_bits(seed, total):
    rot = [[13, 15, 26, 6], [17, 29, 16, 24]]
    ks = [np.uint32(0), np.uint32(seed),
          np.uint32(np.uint32(0) ^ np.uint32(seed) ^ np.uint32(0x1BD11BDA))]
    x0 = np.full(total, ks[0], np.uint32)
    x1 = (np.arange(total, dtype=np.uint32) + ks[1]).astype(np.uint32)

    def rotl(x, d):
        return ((x << np.uint32(d)) | (x >> np.uint32(32 - d))).astype(np.uint32)

    for i in range(5):
        for r in rot[i % 2]:
            x0 = (x0 + x1).astype(np.uint32)
            x1 = rotl(x1, r)
            x1 = x1 ^ x0
        x0 = (x0 + ks[(i + 1) % 3]).astype(np.uint32)
        x1 = (x1 + ks[(i + 2) % 3] + np.uint32(i + 1)).astype(np.uint32)
    return x0 ^ x1


def _erfinv_f32(x):
    x = x.astype(np.float32)
    w = (-np.log(((np.float32(1.0) - x) * (np.float32(1.0) + x))
                 .astype(np.float32)).astype(np.float32))
    small = w < np.float32(5.0)
    ws = (w - np.float32(2.5)).astype(np.float32)
    wb = (np.sqrt(np.maximum(w, np.float32(5.0))).astype(np.float32)
          - np.float32(3.0)).astype(np.float32)
    ca = [2.81022636e-08, 3.43273939e-07, -3.5233877e-06, -4.39150654e-06,
          0.00021858087, -0.00125372503, -0.00417768164, 0.246640727,
          1.50140941]
    cb = [-0.000200214257, 0.000100950558, 0.00134934322, -0.00367342844,
          0.00573950773, -0.0076224613, 0.00943887047, 1.00167406,
          2.83297682]
    pa = np.full_like(ws, np.float32(ca[0]))
    for c in ca[1:]:
        pa = (pa * ws + np.float32(c)).astype(np.float32)
    pb = np.full_like(wb, np.float32(cb[0]))
    for c in cb[1:]:
        pb = (pb * wb + np.float32(c)).astype(np.float32)
    return (np.where(small, pa, pb) * x).astype(np.float32)


def _noise_t(n):
    total = n * _NS * _NW
    bits = _threefry_bits(1, total)
    fb = ((bits >> np.uint32(9)) | np.uint32(0x3F800000)).view(np.float32)
    u01 = fb - np.float32(1.0)
    lo = np.float32(np.nextafter(np.float32(-1.0), np.float32(0.0)))
    u = np.maximum(lo, (u01 * (np.float32(1.0) - lo) + lo).astype(np.float32))
    z = (np.float32(np.sqrt(2.0)) * _erfinv_f32(u)).astype(np.float32)
    z = z.reshape(n // _G, _G, _NS, _NW).transpose(0, 3, 1, 2)
    return z.reshape(n // _G, _NW, _G * _NS).copy()


def _gelu(x):
    return x * 0.5 * (1.0 + jax.lax.erf(x * np.float32(1.0 / np.sqrt(2.0))))


def _masks():
    wm = np.zeros((_NPOS, _NW), np.float32)
    km = np.zeros((_NPOS, _KA * _KA), np.float32)
    for r in range(3):
        for c in range(3):
            j = r * 3 + c
            for dr in range(_KA):
                for dc in range(_KA):
                    p = (10 * r + dr) * _HW + (10 * c + dc)
                    wm[p, j] = 1.0 / (_KA * _KA)
                    km[p, dr * _KA + dc] = 1.0
    return wm, km


def _body(x_ref, noise_ref, sig_ref, w1e_ref, hb_ref,
          w2l_ref, w2g_ref, b2_ref, wm_ref, km_ref, sp_ref, out_ref):
    f32 = jnp.float32
    o96 = jnp.ones((1, _C), f32)
    o784 = jnp.ones((_NPOS, 1), f32)
    c_isq2 = np.float32(1.0 / np.sqrt(2.0))
    wsns = []
    for gidx in range(_G):
        xc = x_ref[0, gidx]
        he = jnp.dot(w1e_ref[...], xc, preferred_element_type=f32)
        m = he[_C:_C + 1, :] * (1.0 / _C)
        q = jnp.dot(o96, xc * xc, preferred_element_type=f32) * (1.0 / _C)
        inv = jax.lax.rsqrt(q - m * m + 1e-5)
        y = he[:_C, :] * inv + hb_ref[...]
        h = y * (1.0 + jax.lax.erf(y * c_isq2))
        glb = jnp.dot(h[_NLOC:, :], o784,
                      preferred_element_type=f32) * (1.0 / _NPOS)
        s = jax.lax.dot_general(w2l_ref[...], h[:_NLOC, :],
                                (((0,), (0,)), ((), ())),
                                preferred_element_type=f32)
        gg = jax.lax.dot_general(w2g_ref[...], glb, (((0,), (0,)), ((), ())),
                                 preferred_element_type=f32)
        s = _gelu(s + gg + b2_ref[...])
        ws = jax.lax.dot_general(wm_ref[...], s, (((0,), (1,)), ((), ())),
                                 preferred_element_type=f32)
        mn = jnp.min(ws, axis=0, keepdims=True)
        mx = jnp.max(ws, axis=0, keepdims=True)
        wsns.append((ws - mn) / (mx - mn + 1e-5))
    wsall = jnp.concatenate(wsns, axis=1)
    pert = (jnp.dot(wsall, sp_ref[...], preferred_element_type=f32)
            + noise_ref[0, 0] * sig_ref[...])
    iota = jax.lax.broadcasted_iota(jnp.int32, (_NW, _G * _NS), 0)
    cmax = jnp.max(pert, axis=0, keepdims=True)
    idx = jnp.min(jnp.where(pert == cmax, iota, _NW), axis=0,
                  keepdims=True)
    inds = jax.lax.dot_general((iota == idx).astype(f32), sp_ref[...],
                               (((1,), (1,)), ((), ())),
                               preferred_element_type=f32) * (1.0 / _NS)
    indrows = jax.lax.dot_general(inds * f32(_KA * _KA), wm_ref[...],
                                  (((0,), (1,)), ((), ())),
                                  preferred_element_type=f32)
    patches = []
    for gidx in range(_G):
        xw = x_ref[0, gidx] * indrows[gidx:gidx + 1, :]
        patches.append(jnp.dot(xw, km_ref[...],
                               preferred_element_type=f32))
    out_ref[0] = jnp.concatenate(patches, axis=1)


def kernel(x, type, H, W, T, sigma, ln_g, ln_b, w1, b1, w2, b2):
    B, Ts = x.shape[0], x.shape[1]
    n = B * Ts
    x4 = x.reshape(B, Ts, _C, _NPOS)
    noise_t = jnp.asarray(
        _noise_t(n).reshape(B, Ts // _G, _NW, _G * _NS))
    sig = jnp.asarray(sigma, jnp.float32).reshape(1, 1)
    wm, km = _masks()
    sp = np.zeros((_G, _G * _NS), np.float32)
    for g in range(_G):
        sp[g, g * _NS:(g + 1) * _NS] = 1.0
    w1g = w1.T * ln_g[None, :]
    w1gc = w1g - jnp.sum(w1g, axis=1, keepdims=True) * (1.0 / _C)
    w1e = jnp.zeros((_C + 8, _C), jnp.float32)
    w1e = w1e.at[:_C].set(w1gc).at[_C].set(1.0)
    hb = (w1.T @ ln_b + b1).reshape(_C, 1)
    out = pl.pallas_call(
        _body,
        grid=(B, Ts // _G),
        in_specs=[
            pl.BlockSpec((1, _G, _C, _NPOS), lambda b, t: (b, t, 0, 0)),
            pl.BlockSpec((1, 1, _NW, _G * _NS), lambda b, t: (b, t, 0, 0)),
            pl.BlockSpec((1, 1), lambda b, t: (0, 0)),
            pl.BlockSpec((_C + 8, _C), lambda b, t: (0, 0)),
            pl.BlockSpec((_C, 1), lambda b, t: (0, 0)),
            pl.BlockSpec((_NLOC, 1), lambda b, t: (0, 0)),
            pl.BlockSpec((_NLOC, 1), lambda b, t: (0, 0)),
            pl.BlockSpec((1, 1), lambda b, t: (0, 0)),
            pl.BlockSpec((_NPOS, _NW), lambda b, t: (0, 0)),
            pl.BlockSpec((_NPOS, _KA * _KA), lambda b, t: (0, 0)),
            pl.BlockSpec((_G, _G * _NS), lambda b, t: (0, 0)),
        ],
        out_specs=pl.BlockSpec((1, _C, _G * _KA * _KA),
                               lambda b, t: (b, 0, t)),
        out_shape=jax.ShapeDtypeStruct((B, _C, Ts * _KA * _KA), jnp.float32),
        compiler_params=pltpu.CompilerParams(
            dimension_semantics=("parallel", "parallel")),
    )(x4, noise_t, sig, w1e, hb,
      0.5 * w2[:_NLOC].reshape(_NLOC, 1), 0.5 * w2[_NLOC:].reshape(_NLOC, 1),
      b2.reshape(1, 1), jnp.asarray(wm), jnp.asarray(km), jnp.asarray(sp))
    return out.reshape(B, _C, Ts, _KA, _KA)
